# baseline (speedup 1.0000x reference)
import functools

import jax
import jax.numpy as jnp
from jax import lax
from jax.experimental import pallas as pl
from jax.experimental.pallas import tpu as pltpu
from jax.experimental.pallas import tpu_sc as plsc

N = 10000
E = 320000
DV = 128
DE = 16
DU = 32
H = 32

NC = 2
NS = 16
NW = NC * NS
C = 256
S = 128
SUB = C // S
NCHUNK = E // C
TMAX = -(-NCHUNK // NW)
NRF = N // NS
EB = 6400

_LOG1P = (0.04106444225260315, -0.15602827499078686, 0.30467224693119505,
          -0.4963682486301464, 0.9998879230599648)


def _softplus_vec(z):
    t = jnp.exp(-jnp.abs(z))
    q = jnp.float32(_LOG1P[0])
    for c in _LOG1P[1:]:
        q = q * t + jnp.float32(c)
    return jnp.maximum(z, jnp.float32(0.0)) + t * q


def _proj_body(nf_ref, wcat_ref, p1_ref, p2_ref, pn_ref):
    p = jnp.dot(nf_ref[...], wcat_ref[...], preferred_element_type=jnp.float32)
    p1_ref[...] = p[:, 0:H]
    p2_ref[...] = p[:, H:2 * H]
    pn_ref[...] = p[:, 2 * H:3 * H]


def _ep_body(eft_ref, wee_ref, u_ref, weu_ref, be_ref, ep_ref):
    ce = jnp.dot(u_ref[...], weu_ref[...], preferred_element_type=jnp.float32) + be_ref[...]
    y = lax.dot_general(eft_ref[...], wee_ref[...], (((0,), (0,)), ((), ())),
                        preferred_element_type=jnp.float32) + ce
    ep_ref[:, :, 0:H] = y.reshape(EB // 8, 8, H)


def _sc_edge_body(src_hbm, dst_hbm, p1_hbm, p2_hbm, ep_hbm, ones_hbm,
                  z32_hbm, z8_hbm,
                  enew_hbm, sums_hbm, cnt_hbm,
                  src_v, dst_v, buf1, buf2, bufp, bufe, ones_v,
                  sums_sp, cnt_sp, sem1, sem2, sem3):
    cid = lax.axis_index("c")
    sid = lax.axis_index("s")
    wid = sid * NC + cid

    pltpu.sync_copy(z32_hbm, sums_sp.at[pl.ds(sid * NRF, NRF)])
    pltpu.sync_copy(z8_hbm, cnt_sp.at[pl.ds(sid * NRF, NRF)])
    pltpu.sync_copy(ones_hbm, ones_v)
    plsc.subcore_barrier()

    def chunk_body(t, carry):
        c = wid + t * NW

        @pl.when(c < NCHUNK)
        def _():
            pltpu.sync_copy(src_hbm.at[pl.ds(c * SUB, SUB)], src_v)
            pltpu.sync_copy(dst_hbm.at[pl.ds(c * SUB, SUB)], dst_v)
            gathers = []
            for j in range(SUB):
                gathers.append(pltpu.async_copy(
                    p1_hbm.at[src_v.at[j]], buf1.at[pl.ds(j * S, S)], sem1))
                gathers.append(pltpu.async_copy(
                    p2_hbm.at[dst_v.at[j]], buf2.at[pl.ds(j * S, S)], sem2))
            dp = pltpu.async_copy(ep_hbm.at[pl.ds(c * (C // 8), C // 8)],
                                  bufp, sem3)
            for g in gathers:
                g.wait()
            dp.wait()

            def row_body(i, c2):
                for jj in range(16):
                    r1 = i * 8 + jj // 2
                    h1 = (jj % 2) * 16
                    re = i * 2 + jj // 8
                    he = (jj % 8) * 16
                    z = (buf1[r1, pl.ds(h1, 16)] + buf2[r1, pl.ds(h1, 16)]
                         + bufp[i, jj // 2, pl.ds(h1, 16)])
                    sp = _softplus_vec(z)
                    buf1[r1, pl.ds(h1, 16)] = sp
                    bufe[re, pl.ds(he, 16)] = sp
                return c2
            lax.fori_loop(0, C * 2 // 16, row_body, 0)

            pltpu.sync_copy(bufe, enew_hbm.at[pl.ds(c * (C // 4), C // 4)])
            for j in range(SUB):
                pltpu.sync_copy(buf1.at[pl.ds(j * S, S)],
                                sums_sp.at[dst_v.at[j]], add=True)
                pltpu.sync_copy(ones_v, cnt_sp.at[dst_v.at[j]], add=True)
        return carry

    lax.fori_loop(0, TMAX, chunk_body, 0)
    plsc.subcore_barrier()

    pltpu.sync_copy(sums_sp.at[pl.ds(sid * NRF, NRF)],
                    sums_hbm.at[cid, pl.ds(sid * NRF, NRF)])
    pltpu.sync_copy(cnt_sp.at[pl.ds(sid * NRF, NRF)],
                    cnt_hbm.at[cid, pl.ds(sid * NRF, NRF)])


_sc_edge = functools.partial(
    pl.kernel,
    out_type=(jax.ShapeDtypeStruct((E // 4, 128), jnp.float32),
              jax.ShapeDtypeStruct((NC, N, H), jnp.float32),
              jax.ShapeDtypeStruct((NC, N, 8), jnp.float32)),
    mesh=plsc.VectorSubcoreMesh(core_axis_name="c", subcore_axis_name="s"),
    compiler_params=pltpu.CompilerParams(use_tc_tiling_on_sc=False),
    scratch_types=(
        pltpu.VMEM((SUB, S), jnp.int32),
        pltpu.VMEM((SUB, S), jnp.int32),
        pltpu.VMEM((C, H), jnp.float32),
        pltpu.VMEM((C, H), jnp.float32),
        pltpu.VMEM((C // 8, 8, 128), jnp.float32),
        pltpu.VMEM((C // 4, 128), jnp.float32),
        pltpu.VMEM((S, 8), jnp.float32),
        pltpu.VMEM_SHARED((N, H), jnp.float32),
        pltpu.VMEM_SHARED((N, 8), jnp.float32),
        pltpu.SemaphoreType.DMA,
        pltpu.SemaphoreType.DMA,
        pltpu.SemaphoreType.DMA,
    ),
)(_sc_edge_body)


def _final_body(sums_ref, cnt_ref, pn_ref, u_ref, wn2_ref, wnu_ref, bn_ref,
                wa1_ref, wa2_ref, wa3_ref, ba_ref, v_ref, u_new_ref):
    def sp(x):
        return jnp.maximum(x, 0.0) + jnp.log(1.0 + jnp.exp(-jnp.abs(x)))

    s = sums_ref[0] + sums_ref[1]
    c8 = cnt_ref[0] + cnt_ref[1]
    cnt = c8[:, 0:1]
    ve = s / jnp.maximum(cnt, 1.0)
    u = u_ref[...]
    cn = jnp.dot(u, wnu_ref[...], preferred_element_type=jnp.float32) + bn_ref[...]
    v_new = sp(pn_ref[...]
               + jnp.dot(ve, wn2_ref[...], preferred_element_type=jnp.float32)
               + cn)
    v_ref[...] = v_new
    ue = jnp.sum(s, axis=0, keepdims=True) * (1.0 / E)
    uv = jnp.sum(v_new, axis=0, keepdims=True) * (1.0 / N)
    u_new_ref[...] = sp(jnp.dot(u, wa1_ref[...], preferred_element_type=jnp.float32)
                        + jnp.dot(ue, wa2_ref[...], preferred_element_type=jnp.float32)
                        + jnp.dot(uv, wa3_ref[...], preferred_element_type=jnp.float32)
                        + ba_ref[...])


def kernel(edge_feat, node_feat, graph_attr, W_e, b_e, W_n, b_n, W_a, b_a,
           edge_index):
    f32 = jnp.float32
    wcat = jnp.concatenate(
        [W_e[0:DV], W_e[DV:2 * DV], W_n[0:DV]], axis=1)
    p1, p2, pn = pl.pallas_call(
        _proj_body,
        out_shape=(jax.ShapeDtypeStruct((N, H), f32),
                   jax.ShapeDtypeStruct((N, H), f32),
                   jax.ShapeDtypeStruct((N, H), f32)),
    )(node_feat, wcat)

    ef_t = edge_feat.T
    ep = pl.pallas_call(
        _ep_body,
        grid=(E // EB,),
        in_specs=[
            pl.BlockSpec((DE, EB), lambda i: (0, i)),
            pl.BlockSpec((DE, H), lambda i: (0, 0)),
            pl.BlockSpec((1, DU), lambda i: (0, 0)),
            pl.BlockSpec((DU, H), lambda i: (0, 0)),
            pl.BlockSpec((1, H), lambda i: (0, 0)),
        ],
        out_specs=pl.BlockSpec((EB // 8, 8, 128), lambda i: (i, 0, 0)),
        out_shape=jax.ShapeDtypeStruct((E // 8, 8, 128), f32),
    )(ef_t, W_e[2 * DV:2 * DV + DE], graph_attr,
      W_e[2 * DV + DE:], b_e.reshape(1, H))

    src = edge_index[0].reshape(E // S, S).astype(jnp.int32)
    dst = edge_index[1].reshape(E // S, S).astype(jnp.int32)
    ones = jnp.ones((S, 8), f32)
    z32 = jnp.zeros((NRF, H), f32)
    z8 = jnp.zeros((NRF, 8), f32)

    e4, sums, cnt = _sc_edge(src, dst, p1, p2, ep, ones, z32, z8)
    e_new = e4.reshape(E, H)

    v_new, u_new = pl.pallas_call(
        _final_body,
        out_shape=(jax.ShapeDtypeStruct((N, H), f32),
                   jax.ShapeDtypeStruct((1, H), f32)),
    )(sums, cnt, pn, graph_attr,
      W_n[DV:DV + H], W_n[DV + H:], b_n.reshape(1, H),
      W_a[0:DU], W_a[DU:DU + H], W_a[DU + H:], b_a.reshape(1, H))

    return (e_new, v_new, u_new)


# speedup vs baseline: 2.2034x; 2.2034x over previous
import functools

import jax
import jax.numpy as jnp
from jax import lax
from jax.experimental import pallas as pl
from jax.experimental.pallas import tpu as pltpu
from jax.experimental.pallas import tpu_sc as plsc

N = 10000
E = 320000
DV = 128
DE = 16
DU = 32
H = 32

NC = 2
NS = 16
NW = NC * NS
C = 256
S = 128
SUB = C // S
NCHUNK = E // C
TMAX = 40
NRF = N // NS
EB = 6400

_LOG1P = (0.04106444225260315, -0.15602827499078686, 0.30467224693119505,
          -0.4963682486301464, 0.9998879230599648)


def _softplus_vec(z):
    t = jnp.exp(-jnp.abs(z))
    q = jnp.float32(_LOG1P[0])
    for c in _LOG1P[1:]:
        q = q * t + jnp.float32(c)
    return jnp.maximum(z, jnp.float32(0.0)) + t * q


def _proj_body(nf_ref, wcat_ref, p1_ref, p2_ref, pn_ref):
    p = jnp.dot(nf_ref[...], wcat_ref[...], preferred_element_type=jnp.float32)
    p1_ref[...] = p[:, 0:H]
    p2_ref[...] = p[:, H:2 * H]
    pn_ref[...] = p[:, 2 * H:3 * H]


def _ep_body(eft_ref, wee_ref, u_ref, weu_ref, be_ref, ep_ref):
    ce = jnp.dot(u_ref[...], weu_ref[...], preferred_element_type=jnp.float32) + be_ref[...]
    y = lax.dot_general(eft_ref[...], wee_ref[...], (((0,), (0,)), ((), ())),
                        preferred_element_type=jnp.float32) + ce
    ep_ref[:, :, 0:H] = y.reshape(EB // 8, 8, H)


def _sc_edge_body(sd_hbm, p1_hbm, p2_hbm, ep_hbm, ones_hbm, z32_hbm, z8_hbm,
                  enew_hbm, sums_hbm, cnt_hbm,
                  idx_v, buf1, buf2, bufp, bufe, ones_v,
                  sums_sp, cnt_sp, semi, sem1, sem2, sem3):
    cid = lax.axis_index("c")
    sid = lax.axis_index("s")
    wid = sid * NC + cid

    pltpu.sync_copy(z32_hbm, sums_sp.at[pl.ds(sid * NRF, NRF)])
    pltpu.sync_copy(z8_hbm, cnt_sp.at[pl.ds(sid * NRF, NRF)])
    pltpu.sync_copy(ones_hbm, ones_v)
    plsc.subcore_barrier()

    def cnum(t):
        return wid + t * NW

    def ep_src(c):
        return ep_hbm.at[pl.ds(c * (C // 8), C // 8), :, pl.ds(0, H)]

    def issue(t, slot):
        c = cnum(t)

        @pl.when(c < NCHUNK)
        def _():
            pltpu.async_copy(sd_hbm.at[:, pl.ds(c * SUB, SUB)],
                             idx_v.at[slot], semi).wait()
            for j in range(SUB):
                pltpu.async_copy(p1_hbm.at[idx_v.at[slot, 0, j]],
                                 buf1.at[pl.ds(slot * C + j * S, S)], sem1)
                pltpu.async_copy(p2_hbm.at[idx_v.at[slot, 1, j]],
                                 buf2.at[pl.ds(slot * C + j * S, S)], sem2)
            pltpu.async_copy(ep_src(c),
                             bufp.at[pl.ds(slot * (C // 8), C // 8)], sem3)

    def wait_loads(t, slot):
        c = cnum(t)

        @pl.when(c < NCHUNK)
        def _():
            for j in range(SUB):
                pltpu.make_async_copy(p1_hbm.at[idx_v.at[slot, 0, j]],
                                      buf1.at[pl.ds(slot * C + j * S, S)],
                                      sem1).wait()
                pltpu.make_async_copy(p2_hbm.at[idx_v.at[slot, 1, j]],
                                      buf2.at[pl.ds(slot * C + j * S, S)],
                                      sem2).wait()
            pltpu.make_async_copy(ep_src(c),
                                  bufp.at[pl.ds(slot * (C // 8), C // 8)],
                                  sem3).wait()

    def work(t, slot):
        c = cnum(t)

        @pl.when(c < NCHUNK)
        def _():
            def row_body(i, c2):
                for jj in range(16):
                    r1 = slot * C + i * 8 + jj // 2
                    h1 = (jj % 2) * 16
                    rp = slot * (C // 8) + i
                    re = i * 2 + jj // 8
                    he = (jj % 8) * 16
                    z = (buf1[r1, pl.ds(h1, 16)] + buf2[r1, pl.ds(h1, 16)]
                         + bufp[rp, jj // 2, pl.ds(h1, 16)])
                    sp = _softplus_vec(z)
                    buf1[r1, pl.ds(h1, 16)] = sp
                    bufe[re, pl.ds(he, 16)] = sp
                return c2
            lax.fori_loop(0, C // 8, row_body, 0)

            pltpu.sync_copy(bufe, enew_hbm.at[pl.ds(c * (C // 4), C // 4)])
            for j in range(SUB):
                pltpu.sync_copy(buf1.at[pl.ds(slot * C + j * S, S)],
                                sums_sp.at[idx_v.at[slot, 1, j]], add=True)
                pltpu.sync_copy(ones_v, cnt_sp.at[idx_v.at[slot, 1, j]],
                                add=True)

    issue(0, 0)

    def loop_body(g, carry):
        t0 = g * 2
        wait_loads(t0, 0)
        issue(t0 + 1, 1)
        work(t0, 0)
        wait_loads(t0 + 1, 1)
        issue(t0 + 2, 0)
        work(t0 + 1, 1)
        return carry

    lax.fori_loop(0, TMAX // 2, loop_body, 0)
    plsc.subcore_barrier()

    pltpu.sync_copy(sums_sp.at[pl.ds(sid * NRF, NRF)],
                    sums_hbm.at[cid, pl.ds(sid * NRF, NRF)])
    pltpu.sync_copy(cnt_sp.at[pl.ds(sid * NRF, NRF)],
                    cnt_hbm.at[cid, pl.ds(sid * NRF, NRF)])


_sc_edge = functools.partial(
    pl.kernel,
    out_type=(jax.ShapeDtypeStruct((E // 4, 128), jnp.float32),
              jax.ShapeDtypeStruct((NC, N, H), jnp.float32),
              jax.ShapeDtypeStruct((NC, N, 8), jnp.float32)),
    mesh=plsc.VectorSubcoreMesh(core_axis_name="c", subcore_axis_name="s"),
    compiler_params=pltpu.CompilerParams(use_tc_tiling_on_sc=False),
    scratch_types=(
        pltpu.VMEM((2, 2, SUB, S), jnp.int32),
        pltpu.VMEM((2 * C, H), jnp.float32),
        pltpu.VMEM((2 * C, H), jnp.float32),
        pltpu.VMEM((2 * (C // 8), 8, H), jnp.float32),
        pltpu.VMEM((C // 4, 128), jnp.float32),
        pltpu.VMEM((S, 8), jnp.float32),
        pltpu.VMEM_SHARED((N, H), jnp.float32),
        pltpu.VMEM_SHARED((N, 8), jnp.float32),
        pltpu.SemaphoreType.DMA,
        pltpu.SemaphoreType.DMA,
        pltpu.SemaphoreType.DMA,
        pltpu.SemaphoreType.DMA,
    ),
)(_sc_edge_body)


def _final_body(sums_ref, cnt_ref, pn_ref, u_ref, wn2_ref, wnu_ref, bn_ref,
                wa1_ref, wa2_ref, wa3_ref, ba_ref, v_ref, u_new_ref):
    def sp(x):
        return jnp.maximum(x, 0.0) + jnp.log(1.0 + jnp.exp(-jnp.abs(x)))

    s = sums_ref[0] + sums_ref[1]
    c8 = cnt_ref[0] + cnt_ref[1]
    cnt = c8[:, 0:1]
    ve = s / jnp.maximum(cnt, 1.0)
    u = u_ref[...]
    cn = jnp.dot(u, wnu_ref[...], preferred_element_type=jnp.float32) + bn_ref[...]
    v_new = sp(pn_ref[...]
               + jnp.dot(ve, wn2_ref[...], preferred_element_type=jnp.float32)
               + cn)
    v_ref[...] = v_new
    ue = jnp.sum(s, axis=0, keepdims=True) * (1.0 / E)
    uv = jnp.sum(v_new, axis=0, keepdims=True) * (1.0 / N)
    u_new_ref[...] = sp(jnp.dot(u, wa1_ref[...], preferred_element_type=jnp.float32)
                        + jnp.dot(ue, wa2_ref[...], preferred_element_type=jnp.float32)
                        + jnp.dot(uv, wa3_ref[...], preferred_element_type=jnp.float32)
                        + ba_ref[...])


def kernel(edge_feat, node_feat, graph_attr, W_e, b_e, W_n, b_n, W_a, b_a,
           edge_index):
    f32 = jnp.float32
    wcat = jnp.concatenate(
        [W_e[0:DV], W_e[DV:2 * DV], W_n[0:DV]], axis=1)
    p1, p2, pn = pl.pallas_call(
        _proj_body,
        out_shape=(jax.ShapeDtypeStruct((N, H), f32),
                   jax.ShapeDtypeStruct((N, H), f32),
                   jax.ShapeDtypeStruct((N, H), f32)),
    )(node_feat, wcat)

    ef_t = edge_feat.T
    ep = pl.pallas_call(
        _ep_body,
        grid=(E // EB,),
        in_specs=[
            pl.BlockSpec((DE, EB), lambda i: (0, i)),
            pl.BlockSpec((DE, H), lambda i: (0, 0)),
            pl.BlockSpec((1, DU), lambda i: (0, 0)),
            pl.BlockSpec((DU, H), lambda i: (0, 0)),
            pl.BlockSpec((1, H), lambda i: (0, 0)),
        ],
        out_specs=pl.BlockSpec((EB // 8, 8, 128), lambda i: (i, 0, 0)),
        out_shape=jax.ShapeDtypeStruct((E // 8, 8, 128), f32),
    )(ef_t, W_e[2 * DV:2 * DV + DE], graph_attr,
      W_e[2 * DV + DE:], b_e.reshape(1, H))

    sd = jnp.stack([edge_index[0].reshape(E // S, S),
                    edge_index[1].reshape(E // S, S)]).astype(jnp.int32)
    ones = jnp.ones((S, 8), f32)
    z32 = jnp.zeros((NRF, H), f32)
    z8 = jnp.zeros((NRF, 8), f32)

    e4, sums, cnt = _sc_edge(sd, p1, p2, ep, ones, z32, z8)
    e_new = e4.reshape(E, H)

    v_new, u_new = pl.pallas_call(
        _final_body,
        out_shape=(jax.ShapeDtypeStruct((N, H), f32),
                   jax.ShapeDtypeStruct((1, H), f32)),
    )(sums, cnt, pn, graph_attr,
      W_n[DV:DV + H], W_n[DV + H:], b_n.reshape(1, H),
      W_a[0:DU], W_a[DU:DU + H], W_a[DU + H:], b_a.reshape(1, H))

    return (e_new, v_new, u_new)


# speedup vs baseline: 2.3791x; 1.0797x over previous
import functools

import jax
import jax.numpy as jnp
from jax import lax
from jax.experimental import pallas as pl
from jax.experimental.pallas import tpu as pltpu
from jax.experimental.pallas import tpu_sc as plsc

N = 10000
E = 320000
DV = 128
DE = 16
DU = 32
H = 32

NC = 2
NS = 16
NW = NC * NS
C = 256
S = 128
SUB = C // S
NCHUNK = E // C
TMAX = 40
NRF = N // NS
EB = 6400

_LOG1P = (0.04106444225260315, -0.15602827499078686, 0.30467224693119505,
          -0.4963682486301464, 0.9998879230599648)


def _softplus_vec(z):
    t = jnp.exp(-jnp.abs(z))
    q = jnp.float32(_LOG1P[0])
    for c in _LOG1P[1:]:
        q = q * t + jnp.float32(c)
    return jnp.maximum(z, jnp.float32(0.0)) + t * q


def _proj_body(nf_ref, wcat_ref, p1_ref, p2_ref, pn_ref):
    p = jnp.dot(nf_ref[...], wcat_ref[...], preferred_element_type=jnp.float32)
    p1_ref[...] = p[:, 0:H]
    p2_ref[...] = p[:, H:2 * H]
    pn_ref[...] = p[:, 2 * H:3 * H]


def _ep_body(eft_ref, wee_ref, u_ref, weu_ref, be_ref, ep_ref):
    ce = jnp.dot(u_ref[...], weu_ref[...], preferred_element_type=jnp.float32) + be_ref[...]
    y = lax.dot_general(eft_ref[...], wee_ref[...], (((0,), (0,)), ((), ())),
                        preferred_element_type=jnp.float32) + ce
    ep_ref[:, :, 0:H] = y.reshape(EB // 8, 8, H)


def _sc_edge_body(sd_hbm, p1_hbm, p2_hbm, ep_hbm, ones_hbm, z32_hbm, z8_hbm,
                  enew_hbm, sums_hbm, cnt_hbm,
                  idx_v, buf1, buf2, bufp, bufe, ones_v,
                  sums_sp, cnt_sp, semi, sem1, sem2, sem3):
    cid = lax.axis_index("c")
    sid = lax.axis_index("s")
    wid = sid * NC + cid

    pltpu.sync_copy(z32_hbm, sums_sp.at[pl.ds(sid * NRF, NRF)])
    pltpu.sync_copy(z8_hbm, cnt_sp.at[pl.ds(sid * NRF, NRF)])
    pltpu.sync_copy(ones_hbm, ones_v)
    plsc.subcore_barrier()

    def cnum(t):
        return wid + t * NW

    def ep_src(c):
        return ep_hbm.at[pl.ds(c * (C // 8), C // 8), :, pl.ds(0, H)]

    def issue(t, slot):
        c = cnum(t)

        @pl.when(c < NCHUNK)
        def _():
            pltpu.async_copy(sd_hbm.at[:, pl.ds(c * SUB, SUB)],
                             idx_v.at[slot], semi).wait()
            for j in range(SUB):
                pltpu.async_copy(p1_hbm.at[idx_v.at[slot, 0, j]],
                                 buf1.at[pl.ds(slot * C + j * S, S)], sem1)
                pltpu.async_copy(p2_hbm.at[idx_v.at[slot, 1, j]],
                                 buf2.at[pl.ds(slot * C + j * S, S)], sem2)
            pltpu.async_copy(ep_src(c),
                             bufp.at[pl.ds(slot * (C // 8), C // 8)], sem3)

    def wait_loads(t, slot):
        c = cnum(t)

        @pl.when(c < NCHUNK)
        def _():
            for j in range(SUB):
                pltpu.make_async_copy(p1_hbm.at[idx_v.at[slot, 0, j]],
                                      buf1.at[pl.ds(slot * C + j * S, S)],
                                      sem1).wait()
                pltpu.make_async_copy(p2_hbm.at[idx_v.at[slot, 1, j]],
                                      buf2.at[pl.ds(slot * C + j * S, S)],
                                      sem2).wait()
            pltpu.make_async_copy(ep_src(c),
                                  bufp.at[pl.ds(slot * (C // 8), C // 8)],
                                  sem3).wait()

    def work(t, slot):
        c = cnum(t)

        @pl.when(c < NCHUNK)
        def _():
            def row_body(i, c2):
                for jj in range(16):
                    r1 = slot * C + i * 8 + jj // 2
                    h1 = (jj % 2) * 16
                    rp = slot * (C // 8) + i
                    re = i * 2 + jj // 8
                    he = (jj % 8) * 16
                    z = (buf1[r1, pl.ds(h1, 16)] + buf2[r1, pl.ds(h1, 16)]
                         + bufp[rp, jj // 2, pl.ds(h1, 16)])
                    sp = _softplus_vec(z)
                    buf1[r1, pl.ds(h1, 16)] = sp
                    bufe[re, pl.ds(he, 16)] = sp
                return c2
            lax.fori_loop(0, C // 8, row_body, 0)

            pltpu.sync_copy(bufe, enew_hbm.at[pl.ds(c * (C // 4), C // 4)])
            for j in range(SUB):
                pltpu.sync_copy(buf1.at[pl.ds(slot * C + j * S, S)],
                                sums_sp.at[idx_v.at[slot, 1, j]], add=True)
                pltpu.sync_copy(ones_v, cnt_sp.at[idx_v.at[slot, 1, j]],
                                add=True)

    issue(0, 0)

    def loop_body(g, carry):
        t0 = g * 2
        wait_loads(t0, 0)
        issue(t0 + 1, 1)
        work(t0, 0)
        wait_loads(t0 + 1, 1)
        issue(t0 + 2, 0)
        work(t0 + 1, 1)
        return carry

    lax.fori_loop(0, TMAX // 2, loop_body, 0)
    plsc.subcore_barrier()

    pltpu.sync_copy(sums_sp.at[pl.ds(sid * NRF, NRF)],
                    sums_hbm.at[cid, pl.ds(sid * NRF, NRF)])
    pltpu.sync_copy(cnt_sp.at[pl.ds(sid * NRF, NRF)],
                    cnt_hbm.at[cid, pl.ds(sid * NRF, NRF)])


_sc_edge = functools.partial(
    pl.kernel,
    out_type=(jax.ShapeDtypeStruct((E // 4, 128), jnp.float32),
              jax.ShapeDtypeStruct((NC, N, H), jnp.float32),
              jax.ShapeDtypeStruct((NC, N, 8), jnp.float32)),
    mesh=plsc.VectorSubcoreMesh(core_axis_name="c", subcore_axis_name="s"),
    compiler_params=pltpu.CompilerParams(use_tc_tiling_on_sc=False),
    scratch_types=(
        pltpu.VMEM((2, 2, SUB, S), jnp.int32),
        pltpu.VMEM((2 * C, H), jnp.float32),
        pltpu.VMEM((2 * C, H), jnp.float32),
        pltpu.VMEM((2 * (C // 8), 8, H), jnp.float32),
        pltpu.VMEM((C // 4, 128), jnp.float32),
        pltpu.VMEM((S, 8), jnp.float32),
        pltpu.VMEM_SHARED((N, H), jnp.float32),
        pltpu.VMEM_SHARED((N, 8), jnp.float32),
        pltpu.SemaphoreType.DMA,
        pltpu.SemaphoreType.DMA,
        pltpu.SemaphoreType.DMA,
        pltpu.SemaphoreType.DMA,
    ),
)(_sc_edge_body)


def _final_body(sums_ref, cnt_ref, pn_ref, u_ref, wn2_ref, wnu_ref, bn_ref,
                wa1_ref, wa2_ref, wa3_ref, ba_ref, v_ref, u_new_ref):
    def sp(x):
        return jnp.maximum(x, 0.0) + jnp.log(1.0 + jnp.exp(-jnp.abs(x)))

    s = sums_ref[0] + sums_ref[1]
    c8 = cnt_ref[0] + cnt_ref[1]
    cnt = c8[:, 0:1]
    ve = s / jnp.maximum(cnt, 1.0)
    u = u_ref[...]
    cn = jnp.dot(u, wnu_ref[...], preferred_element_type=jnp.float32) + bn_ref[...]
    v_new = sp(pn_ref[...]
               + jnp.dot(ve, wn2_ref[...], preferred_element_type=jnp.float32)
               + cn)
    v_ref[...] = v_new
    ue = jnp.sum(s, axis=0, keepdims=True) * (1.0 / E)
    uv = jnp.sum(v_new, axis=0, keepdims=True) * (1.0 / N)
    u_new_ref[...] = sp(jnp.dot(u, wa1_ref[...], preferred_element_type=jnp.float32)
                        + jnp.dot(ue, wa2_ref[...], preferred_element_type=jnp.float32)
                        + jnp.dot(uv, wa3_ref[...], preferred_element_type=jnp.float32)
                        + ba_ref[...])


def kernel(edge_feat, node_feat, graph_attr, W_e, b_e, W_n, b_n, W_a, b_a,
           edge_index):
    f32 = jnp.float32
    wcat = jnp.concatenate(
        [W_e[0:DV], W_e[DV:2 * DV], W_n[0:DV]], axis=1)
    p1, p2, pn = pl.pallas_call(
        _proj_body,
        out_shape=(jax.ShapeDtypeStruct((N, H), f32),
                   jax.ShapeDtypeStruct((N, H), f32),
                   jax.ShapeDtypeStruct((N, H), f32)),
    )(node_feat, wcat)

    ef_t = edge_feat.T
    ep = pl.pallas_call(
        _ep_body,
        grid=(E // EB,),
        in_specs=[
            pl.BlockSpec((DE, EB), lambda i: (0, i)),
            pl.BlockSpec((DE, H), lambda i: (0, 0)),
            pl.BlockSpec((1, DU), lambda i: (0, 0)),
            pl.BlockSpec((DU, H), lambda i: (0, 0)),
            pl.BlockSpec((1, H), lambda i: (0, 0)),
        ],
        out_specs=pl.BlockSpec((EB // 8, 8, 128), lambda i: (i, 0, 0)),
        out_shape=jax.ShapeDtypeStruct((E // 8, 8, 128), f32),
    )(ef_t, W_e[2 * DV:2 * DV + DE], graph_attr,
      W_e[2 * DV + DE:], b_e.reshape(1, H))

    sd = jnp.stack([edge_index[0].reshape(E // S, S),
                    edge_index[1].reshape(E // S, S)]).astype(jnp.int32)
    ones = jnp.ones((S, 8), f32)
    z32 = jnp.zeros((NRF, H), f32)
    z8 = jnp.zeros((NRF, 8), f32)

    e4, sums, cnt = _sc_edge(sd, p1, p2, ep, ones, z32, z8)
    e_new = e4.reshape(E // 4, 4, H).transpose(2, 0, 1).reshape(H, E).T

    v_new, u_new = pl.pallas_call(
        _final_body,
        out_shape=(jax.ShapeDtypeStruct((N, H), f32),
                   jax.ShapeDtypeStruct((1, H), f32)),
    )(sums, cnt, pn, graph_attr,
      W_n[DV:DV + H], W_n[DV + H:], b_n.reshape(1, H),
      W_a[0:DU], W_a[DU:DU + H], W_a[DU + H:], b_a.reshape(1, H))

    return (e_new, v_new, u_new)


# speedup vs baseline: 2.4433x; 1.0270x over previous
import functools

import jax
import jax.numpy as jnp
from jax import lax
from jax.experimental import pallas as pl
from jax.experimental.pallas import tpu as pltpu
from jax.experimental.pallas import tpu_sc as plsc

N = 10000
E = 320000
DV = 128
DE = 16
DU = 32
H = 32

NC = 2
NS = 16
NW = NC * NS
C = 256
S = 128
SUB = C // S
NCHUNK = E // C
TMAX = 40
NRF = N // NS
EB = 6400

_LOG1P = (0.04106444225260315, -0.15602827499078686, 0.30467224693119505,
          -0.4963682486301464, 0.9998879230599648)


def _softplus_vec(z):
    t = jnp.exp(-jnp.abs(z))
    q = jnp.float32(_LOG1P[0])
    for c in _LOG1P[1:]:
        q = q * t + jnp.float32(c)
    return jnp.maximum(z, jnp.float32(0.0)) + t * q


def _proj_body(nf_ref, wcat_ref, p1_ref, p2_ref, pn_ref):
    p = jnp.dot(nf_ref[...], wcat_ref[...], preferred_element_type=jnp.float32)
    p1_ref[...] = p[:, 0:H]
    p2_ref[...] = p[:, H:2 * H]
    pn_ref[...] = p[:, 2 * H:3 * H]


def _ep_body(eft_ref, wee_ref, u_ref, weu_ref, be_ref, ep_ref):
    ce = jnp.dot(u_ref[...], weu_ref[...], preferred_element_type=jnp.float32) + be_ref[...]
    y = lax.dot_general(eft_ref[...], wee_ref[...], (((0,), (0,)), ((), ())),
                        preferred_element_type=jnp.float32) + ce
    ep_ref[:, :, 0:H] = y.reshape(EB // 8, 8, H)


def _sc_edge_body(sd_hbm, p1_hbm, p2_hbm, ep_hbm, ones_hbm, z32_hbm, z8_hbm,
                  enew_hbm, sums_hbm, cnt_hbm,
                  idx_v, buf1, buf2, bufp, bufe, ones_v,
                  sums_sp, cnt_sp, semi, sem1, sem2, sem3, sem4):
    cid = lax.axis_index("c")
    sid = lax.axis_index("s")
    wid = sid * NC + cid

    pltpu.sync_copy(z32_hbm, sums_sp.at[pl.ds(sid * NRF, NRF)])
    pltpu.sync_copy(z8_hbm, cnt_sp.at[pl.ds(sid * NRF, NRF)])
    pltpu.sync_copy(ones_hbm, ones_v)
    plsc.subcore_barrier()

    def cnum(t):
        return wid + t * NW

    def ep_src(c):
        return ep_hbm.at[pl.ds(c * (C // 8), C // 8), :, pl.ds(0, H)]

    def issue(t, slot):
        c = cnum(t)

        @pl.when(c < NCHUNK)
        def _():
            pltpu.async_copy(sd_hbm.at[:, pl.ds(c * SUB, SUB)],
                             idx_v.at[slot], semi).wait()
            for j in range(SUB):
                pltpu.async_copy(p1_hbm.at[idx_v.at[slot, 0, j]],
                                 buf1.at[pl.ds(slot * C + j * S, S)], sem1)
                pltpu.async_copy(p2_hbm.at[idx_v.at[slot, 1, j]],
                                 buf2.at[pl.ds(slot * C + j * S, S)], sem2)
            pltpu.async_copy(ep_src(c),
                             bufp.at[pl.ds(slot * (C // 8), C // 8)], sem3)

    def wait_loads(t, slot):
        c = cnum(t)

        @pl.when(c < NCHUNK)
        def _():
            for j in range(SUB):
                pltpu.make_async_copy(p1_hbm.at[idx_v.at[slot, 0, j]],
                                      buf1.at[pl.ds(slot * C + j * S, S)],
                                      sem1).wait()
                pltpu.make_async_copy(p2_hbm.at[idx_v.at[slot, 1, j]],
                                      buf2.at[pl.ds(slot * C + j * S, S)],
                                      sem2).wait()
            pltpu.make_async_copy(ep_src(c),
                                  bufp.at[pl.ds(slot * (C // 8), C // 8)],
                                  sem3).wait()

    def work(t, slot):
        c = cnum(t)

        @pl.when(c < NCHUNK)
        def _():
            c_pp = c - 2 * NW

            @pl.when(c_pp >= 0)
            def _():
                pltpu.make_async_copy(
                    bufe.at[pl.ds(slot * (C // 4), C // 4)],
                    enew_hbm.at[pl.ds(c_pp * (C // 4), C // 4)],
                    sem4).wait()

            def row_body(i, c2):
                for jj in range(16):
                    r1 = slot * C + i * 8 + jj // 2
                    h1 = (jj % 2) * 16
                    rp = slot * (C // 8) + i
                    re = slot * (C // 4) + i * 2 + jj // 8
                    he = (jj % 8) * 16
                    z = (buf1[r1, pl.ds(h1, 16)] + buf2[r1, pl.ds(h1, 16)]
                         + bufp[rp, jj // 2, pl.ds(h1, 16)])
                    sp = _softplus_vec(z)
                    buf1[r1, pl.ds(h1, 16)] = sp
                    bufe[re, pl.ds(he, 16)] = sp
                return c2
            lax.fori_loop(0, C // 8, row_body, 0)

            pltpu.async_copy(bufe.at[pl.ds(slot * (C // 4), C // 4)],
                             enew_hbm.at[pl.ds(c * (C // 4), C // 4)], sem4)
            for j in range(SUB):
                pltpu.sync_copy(buf1.at[pl.ds(slot * C + j * S, S)],
                                sums_sp.at[idx_v.at[slot, 1, j]], add=True)
                pltpu.sync_copy(ones_v, cnt_sp.at[idx_v.at[slot, 1, j]],
                                add=True)

    issue(0, 0)

    def loop_body(g, carry):
        t0 = g * 2
        wait_loads(t0, 0)
        issue(t0 + 1, 1)
        work(t0, 0)
        wait_loads(t0 + 1, 1)
        issue(t0 + 2, 0)
        work(t0 + 1, 1)
        return carry

    lax.fori_loop(0, TMAX // 2, loop_body, 0)
    for slot in range(2):
        pltpu.make_async_copy(bufe.at[pl.ds(slot * (C // 4), C // 4)],
                              enew_hbm.at[pl.ds(0, C // 4)], sem4).wait()
    plsc.subcore_barrier()

    pltpu.sync_copy(sums_sp.at[pl.ds(sid * NRF, NRF)],
                    sums_hbm.at[cid, pl.ds(sid * NRF, NRF)])
    pltpu.sync_copy(cnt_sp.at[pl.ds(sid * NRF, NRF)],
                    cnt_hbm.at[cid, pl.ds(sid * NRF, NRF)])


_sc_edge = functools.partial(
    pl.kernel,
    out_type=(jax.ShapeDtypeStruct((E // 4, 128), jnp.float32),
              jax.ShapeDtypeStruct((NC, N, H), jnp.float32),
              jax.ShapeDtypeStruct((NC, N, 8), jnp.float32)),
    mesh=plsc.VectorSubcoreMesh(core_axis_name="c", subcore_axis_name="s"),
    compiler_params=pltpu.CompilerParams(use_tc_tiling_on_sc=False),
    scratch_types=(
        pltpu.VMEM((2, 2, SUB, S), jnp.int32),
        pltpu.VMEM((2 * C, H), jnp.float32),
        pltpu.VMEM((2 * C, H), jnp.float32),
        pltpu.VMEM((2 * (C // 8), 8, H), jnp.float32),
        pltpu.VMEM((2 * (C // 4), 128), jnp.float32),
        pltpu.VMEM((S, 8), jnp.float32),
        pltpu.VMEM_SHARED((N, H), jnp.float32),
        pltpu.VMEM_SHARED((N, 8), jnp.float32),
        pltpu.SemaphoreType.DMA,
        pltpu.SemaphoreType.DMA,
        pltpu.SemaphoreType.DMA,
        pltpu.SemaphoreType.DMA,
        pltpu.SemaphoreType.DMA,
    ),
)(_sc_edge_body)


def _final_body(sums_ref, cnt_ref, pn_ref, u_ref, wn2_ref, wnu_ref, bn_ref,
                wa1_ref, wa2_ref, wa3_ref, ba_ref, v_ref, u_new_ref):
    def sp(x):
        return jnp.maximum(x, 0.0) + jnp.log(1.0 + jnp.exp(-jnp.abs(x)))

    s = sums_ref[0] + sums_ref[1]
    c8 = cnt_ref[0] + cnt_ref[1]
    cnt = c8[:, 0:1]
    ve = s / jnp.maximum(cnt, 1.0)
    u = u_ref[...]
    cn = jnp.dot(u, wnu_ref[...], preferred_element_type=jnp.float32) + bn_ref[...]
    v_new = sp(pn_ref[...]
               + jnp.dot(ve, wn2_ref[...], preferred_element_type=jnp.float32)
               + cn)
    v_ref[...] = v_new
    ue = jnp.sum(s, axis=0, keepdims=True) * (1.0 / E)
    uv = jnp.sum(v_new, axis=0, keepdims=True) * (1.0 / N)
    u_new_ref[...] = sp(jnp.dot(u, wa1_ref[...], preferred_element_type=jnp.float32)
                        + jnp.dot(ue, wa2_ref[...], preferred_element_type=jnp.float32)
                        + jnp.dot(uv, wa3_ref[...], preferred_element_type=jnp.float32)
                        + ba_ref[...])


def kernel(edge_feat, node_feat, graph_attr, W_e, b_e, W_n, b_n, W_a, b_a,
           edge_index):
    f32 = jnp.float32
    wcat = jnp.concatenate(
        [W_e[0:DV], W_e[DV:2 * DV], W_n[0:DV]], axis=1)
    p1, p2, pn = pl.pallas_call(
        _proj_body,
        out_shape=(jax.ShapeDtypeStruct((N, H), f32),
                   jax.ShapeDtypeStruct((N, H), f32),
                   jax.ShapeDtypeStruct((N, H), f32)),
    )(node_feat, wcat)

    ef_t = edge_feat.T
    ep = pl.pallas_call(
        _ep_body,
        grid=(E // EB,),
        in_specs=[
            pl.BlockSpec((DE, EB), lambda i: (0, i)),
            pl.BlockSpec((DE, H), lambda i: (0, 0)),
            pl.BlockSpec((1, DU), lambda i: (0, 0)),
            pl.BlockSpec((DU, H), lambda i: (0, 0)),
            pl.BlockSpec((1, H), lambda i: (0, 0)),
        ],
        out_specs=pl.BlockSpec((EB // 8, 8, 128), lambda i: (i, 0, 0)),
        out_shape=jax.ShapeDtypeStruct((E // 8, 8, 128), f32),
    )(ef_t, W_e[2 * DV:2 * DV + DE], graph_attr,
      W_e[2 * DV + DE:], b_e.reshape(1, H))

    sd = jnp.stack([edge_index[0].reshape(E // S, S),
                    edge_index[1].reshape(E // S, S)]).astype(jnp.int32)
    ones = jnp.ones((S, 8), f32)
    z32 = jnp.zeros((NRF, H), f32)
    z8 = jnp.zeros((NRF, 8), f32)

    e4, sums, cnt = _sc_edge(sd, p1, p2, ep, ones, z32, z8)
    e_new = e4.reshape(E // 4, 4, H).transpose(2, 0, 1).reshape(H, E).T

    v_new, u_new = pl.pallas_call(
        _final_body,
        out_shape=(jax.ShapeDtypeStruct((N, H), f32),
                   jax.ShapeDtypeStruct((1, H), f32)),
    )(sums, cnt, pn, graph_attr,
      W_n[DV:DV + H], W_n[DV + H:], b_n.reshape(1, H),
      W_a[0:DU], W_a[DU:DU + H], W_a[DU + H:], b_a.reshape(1, H))

    return (e_new, v_new, u_new)
